# baseline (device time: 20551 ns/iter reference)
import jax
import jax.numpy as jnp
from jax import lax
from jax.experimental import pallas as pl
from jax.experimental.pallas import tpu as pltpu

N_DEV = 8
LOG_N = 3
EXPERTS_PER_DEV = 2


def kernel(x, router_W, route_idx, expert_W):
    del router_W
    n, d = x.shape
    h = expert_W.shape[-1]

    def body(x_ref, idx_ref, ew_ref, out_ref, comm_ref, send_sems, recv_sems):
        my = lax.axis_index("i")

        barrier_sem = pltpu.get_barrier_semaphore()
        for k in range(LOG_N):
            pl.semaphore_signal(
                barrier_sem, inc=1,
                device_id=(my ^ (1 << k),),
                device_id_type=pl.DeviceIdType.MESH,
            )

        idx = idx_ref[:, :]
        acc = jnp.zeros((n, h), jnp.float32)
        for e in range(EXPERTS_PER_DEV):
            ge = my * EXPERTS_PER_DEV + e
            y = jnp.dot(x_ref[:, :], ew_ref[e, :, :],
                        preferred_element_type=jnp.float32)
            acc = acc + jnp.where(idx == ge, y, 0.0)
        out_ref[:, :] = acc

        pl.semaphore_wait(barrier_sem, LOG_N)

        for k in range(LOG_N):
            partner = my ^ (1 << k)
            rdma = pltpu.make_async_remote_copy(
                src_ref=out_ref,
                dst_ref=comm_ref.at[k],
                send_sem=send_sems.at[k],
                recv_sem=recv_sems.at[k],
                device_id=(partner,),
                device_id_type=pl.DeviceIdType.MESH,
            )
            rdma.start()
            rdma.wait()
            out_ref[:, :] = out_ref[:, :] + comm_ref[k, :, :]

    return pl.pallas_call(
        body,
        out_shape=jax.ShapeDtypeStruct((n, h), jnp.float32),
        in_specs=[
            pl.BlockSpec(memory_space=pltpu.VMEM),
            pl.BlockSpec(memory_space=pltpu.VMEM),
            pl.BlockSpec(memory_space=pltpu.VMEM),
        ],
        out_specs=pl.BlockSpec(memory_space=pltpu.VMEM),
        scratch_shapes=[
            pltpu.VMEM((LOG_N, n, h), jnp.float32),
            pltpu.SemaphoreType.DMA((LOG_N,)),
            pltpu.SemaphoreType.DMA((LOG_N,)),
        ],
        compiler_params=pltpu.CompilerParams(collective_id=0),
    )(x, route_idx, expert_W)
